# baseline (speedup 1.0000x reference)
import functools

import jax
import jax.numpy as jnp
from jax import lax
from jax.experimental import pallas as pl
from jax.experimental.pallas import tpu as pltpu
from jax.experimental.pallas import tpu_sc as plsc

N_NODES = 10000
N_GRAPHS = 100
HID = 128
NC = 2
NS = 16
NW = NC * NS
E = 320000
EH = E // 2
EPT = EH // NW
CHUNK = 40
NCHUNK = EPT // CHUNK
BE = 8000
NBLK = EH // BE

_PREC = lax.Precision.HIGHEST
_PREC_EDGE = lax.Precision.DEFAULT


@functools.cache
def _make_sc_gather():
    mesh = plsc.VectorSubcoreMesh(core_axis_name="c", subcore_axis_name="s",
                                  num_cores=NC, num_subcores=NS)

    @functools.partial(
        pl.kernel, mesh=mesh,
        out_type=jax.ShapeDtypeStruct((EH, HID), jnp.float32),
        scratch_types=[
            pltpu.VMEM((NCHUNK, CHUNK), jnp.int32),
            pltpu.VMEM((CHUNK, HID), jnp.float32),
            pltpu.VMEM((CHUNK, HID), jnp.float32),
            pltpu.VMEM_SHARED((N_NODES, HID), jnp.float32),
            pltpu.SemaphoreType.DMA,
            pltpu.SemaphoreType.DMA,
            pltpu.SemaphoreType.DMA,
            pltpu.SemaphoreType.DMA,
        ],
    )
    def gather_k(table_hbm, idx_hbm, out_hbm,
                 idx_v, rows0, rows1, ctab, s0, s1, w0, w1):
        sid = lax.axis_index("s")
        wid = sid * NC + lax.axis_index("c")
        rows = (rows0, rows1)
        ssem = (s0, s1)
        wsem = (w0, w1)
        rpt = 624
        rem = N_NODES - NS * rpt
        pltpu.sync_copy(table_hbm.at[pl.ds(sid * rpt, rpt)],
                        ctab.at[pl.ds(sid * rpt, rpt)])

        @pl.when(sid == 0)
        def _():
            pltpu.sync_copy(table_hbm.at[pl.ds(NS * rpt, rem)],
                            ctab.at[pl.ds(NS * rpt, rem)])

        pltpu.sync_copy(idx_hbm.at[wid], idx_v)
        plsc.subcore_barrier()

        def out_slice(g):
            return out_hbm.at[pl.ds(wid * EPT + g * CHUNK, CHUNK)]

        def drain(sem, buf):
            pltpu.make_async_copy(out_hbm.at[pl.ds(0, CHUNK)], buf, sem).wait()

        pltpu.async_copy(ctab.at[idx_v.at[0]], rows[0], ssem[0])

        def body(t, _):
            for i, b in ((0, 1), (1, 0)):
                g = 2 * t + 1 + i
                if i == 0:
                    @pl.when(t > 0)
                    def _():
                        drain(wsem[b], rows[b])
                else:
                    drain(wsem[b], rows[b])
                pltpu.async_copy(ctab.at[idx_v.at[g]], rows[b], ssem[b])
                b1 = 1 - b
                drain(ssem[b1], rows[b1])
                pltpu.async_copy(rows[b1], out_slice(g - 1), wsem[b1])
            return 0

        lax.fori_loop(0, (NCHUNK - 1) // 2, body, 0)
        last = NCHUNK - 1
        drain(ssem[0], rows[0])
        pltpu.async_copy(rows[0], out_slice(last), wsem[0])
        drain(wsem[1], rows[1])
        drain(wsem[0], rows[0])

    return gather_k


@functools.cache
def _make_sc_scatter():
    mesh = plsc.VectorSubcoreMesh(core_axis_name="c", subcore_axis_name="s",
                                  num_cores=NC, num_subcores=NS)

    @functools.partial(
        pl.kernel, mesh=mesh,
        out_type=jax.ShapeDtypeStruct((NC, N_NODES, HID), jnp.float32),
        scratch_types=[
            pltpu.VMEM((NCHUNK, CHUNK), jnp.int32),
            pltpu.VMEM((CHUNK, HID), jnp.float32),
            pltpu.VMEM((CHUNK, HID), jnp.float32),
            pltpu.VMEM_SHARED((N_NODES, HID), jnp.float32),
            pltpu.SemaphoreType.DMA,
            pltpu.SemaphoreType.DMA,
            pltpu.SemaphoreType.DMA,
            pltpu.SemaphoreType.DMA,
        ],
    )
    def scatter_k(rows_hbm, idx_hbm, zero_hbm, out_hbm,
                  idx_v, rows0, rows1, shared, r0, r1, c0, c1):
        cid = lax.axis_index("c")
        sid = lax.axis_index("s")
        wid = sid * NC + cid
        rows = (rows0, rows1)
        rsem = (r0, r1)
        csem = (c0, c1)
        rpt = 624
        rem = N_NODES - NS * rpt
        pltpu.sync_copy(zero_hbm.at[pl.ds(sid * rpt, rpt)],
                        shared.at[pl.ds(sid * rpt, rpt)])

        @pl.when(sid == 0)
        def _():
            pltpu.sync_copy(zero_hbm.at[pl.ds(NS * rpt, rem)],
                            shared.at[pl.ds(NS * rpt, rem)])

        pltpu.sync_copy(idx_hbm.at[wid], idx_v)

        def in_slice(g):
            return rows_hbm.at[pl.ds(wid * EPT + g * CHUNK, CHUNK)]

        def drain(sem, buf):
            pltpu.make_async_copy(rows_hbm.at[pl.ds(0, CHUNK)], buf, sem).wait()

        pltpu.async_copy(in_slice(0), rows[0], rsem[0])
        plsc.subcore_barrier()

        def body(t, _):
            for i, b in ((0, 1), (1, 0)):
                g = 2 * t + 1 + i
                if i == 0:
                    @pl.when(t > 0)
                    def _():
                        drain(csem[b], rows[b])
                else:
                    drain(csem[b], rows[b])
                pltpu.async_copy(in_slice(g), rows[b], rsem[b])
                b1 = 1 - b
                drain(rsem[b1], rows[b1])
                pltpu.async_copy(rows[b1], shared.at[idx_v.at[g - 1]],
                                 csem[b1], add=True)
            return 0

        lax.fori_loop(0, (NCHUNK - 1) // 2, body, 0)
        last = NCHUNK - 1
        drain(rsem[0], rows[0])
        pltpu.async_copy(rows[0], shared.at[idx_v.at[last]], csem[0], add=True)
        drain(csem[1], rows[1])
        drain(csem[0], rows[0])
        plsc.subcore_barrier()
        pltpu.sync_copy(shared.at[pl.ds(sid * rpt, rpt)],
                        out_hbm.at[cid, pl.ds(sid * rpt, rpt)])

        @pl.when(sid == 0)
        def _():
            pltpu.sync_copy(shared.at[pl.ds(NS * rpt, rem)],
                            out_hbm.at[cid, pl.ds(NS * rpt, rem)])

    return scatter_k


def _sc_gather(table, idx2):
    return _make_sc_gather()(table, idx2)


def _sc_scatter(rows, idx2, zeros_tbl):
    return _make_sc_scatter()(rows, idx2, zeros_tbl)


def _node_matmul(x, w):
    def body(x_ref, w_ref, o_ref):
        o_ref[...] = jnp.dot(x_ref[...], w_ref[...],
                             preferred_element_type=jnp.float32,
                             precision=_PREC)
    return pl.pallas_call(
        body,
        out_shape=jax.ShapeDtypeStruct((x.shape[0], w.shape[1]), jnp.float32),
    )(x, w)


def _combine(a, ta, tb):
    def body(a_ref, ta_ref, tb_ref, o_ref):
        o_ref[...] = (a_ref[...] + ta_ref[0] + ta_ref[1]
                      + tb_ref[0] + tb_ref[1])
    return pl.pallas_call(
        body,
        out_shape=jax.ShapeDtypeStruct((N_NODES, HID), jnp.float32),
    )(a, ta, tb)


def _edge_pass(g, he, w_i2, w_h, r_other, emit_r):
    has_prev = r_other is not None

    def body(*refs):
        if has_prev:
            g_ref, he_ref, wi_ref, wh_ref, r_ref, o_ref = refs
        else:
            g_ref, he_ref, wi_ref, wh_ref, o_ref = refs
        acc = g_ref[...] + jnp.dot(he_ref[...], wi_ref[...],
                                   preferred_element_type=jnp.float32,
                                   precision=_PREC_EDGE)
        if has_prev:
            acc = acc - r_ref[...]
        msg = jnp.maximum(acc, 0.0)
        if emit_r:
            o_ref[...] = jnp.dot(msg, wh_ref[...],
                                 preferred_element_type=jnp.float32,
                                 precision=_PREC_EDGE)
        else:
            o_ref[...] = msg

    in_specs = [
        pl.BlockSpec((BE, HID), lambda b: (b, 0)),
        pl.BlockSpec((BE, 16), lambda b: (b, 0)),
        pl.BlockSpec((16, HID), lambda b: (0, 0)),
        pl.BlockSpec((HID, HID), lambda b: (0, 0)),
    ]
    args = [g, he, w_i2, w_h]
    if has_prev:
        in_specs.append(pl.BlockSpec((BE, HID), lambda b: (b, 0)))
        args.append(r_other)
    return pl.pallas_call(
        body,
        grid=(NBLK,),
        in_specs=in_specs,
        out_specs=pl.BlockSpec((BE, HID), lambda b: (b, 0)),
        out_shape=jax.ShapeDtypeStruct((EH, HID), jnp.float32),
    )(*args)


def _final(hv, ma, mb, w_o1, w_o2, b_o, gids):
    def body(hv_ref, ma_ref, mb_ref, w1_ref, w2_ref, b_ref, gid_ref, o_ref):
        m = ma_ref[0] + ma_ref[1] + mb_ref[0] + mb_ref[1]
        h = jnp.dot(hv_ref[...], w1_ref[...],
                    preferred_element_type=jnp.float32, precision=_PREC)
        h += jnp.dot(m, w2_ref[...],
                     preferred_element_type=jnp.float32, precision=_PREC)
        h = jnp.maximum(h + b_ref[...], 0.0)
        rows = lax.broadcasted_iota(jnp.int32, (N_GRAPHS, N_NODES), 0)
        p = (gid_ref[...] == rows).astype(jnp.float32)
        g_sum = jnp.dot(p, h, preferred_element_type=jnp.float32,
                        precision=_PREC)
        counts = jnp.maximum(jnp.sum(p, axis=1, keepdims=True), 1.0)
        o_ref[...] = g_sum / counts

    return pl.pallas_call(
        body,
        out_shape=jax.ShapeDtypeStruct((N_GRAPHS, HID), jnp.float32),
    )(hv, ma, mb, w_o1, w_o2, b_o, gids)


def kernel(hv, edge_index, he, graph_ids, W_i, W_h, W_o, b_o):
    u, v = edge_index[0], edge_index[1]
    s1 = u.reshape(NW, NCHUNK, CHUNK)
    s2 = v.reshape(NW, NCHUNK, CHUNK)
    d1 = s2
    d2 = s1
    he1 = he[:EH]
    he2 = he[EH:]
    zeros_tbl = jnp.zeros((N_NODES, HID), jnp.float32)

    atom = hv.shape[1]
    w_i1 = W_i[:atom]
    w_i2 = W_i[atom:]
    w_o1 = W_o[:atom]
    w_o2 = W_o[atom:]
    b2 = b_o.reshape(1, HID)
    gid2 = graph_ids.reshape(1, N_NODES)

    a_tbl = _node_matmul(hv, w_i1)

    g1 = _sc_gather(a_tbl, s1)
    r1 = _edge_pass(g1, he1, w_i2, W_h, None, True)
    g2 = _sc_gather(a_tbl, s2)
    r2 = _edge_pass(g2, he2, w_i2, W_h, None, True)
    for _ in range(4):
        ta = _sc_scatter(r1, d1, zeros_tbl)
        tb = _sc_scatter(r2, d2, zeros_tbl)
        c = _combine(a_tbl, ta, tb)
        g1 = _sc_gather(c, s1)
        n1 = _edge_pass(g1, he1, w_i2, W_h, r2, True)
        g2 = _sc_gather(c, s2)
        n2 = _edge_pass(g2, he2, w_i2, W_h, r1, True)
        r1, r2 = n1, n2
    ta = _sc_scatter(r1, d1, zeros_tbl)
    tb = _sc_scatter(r2, d2, zeros_tbl)
    c = _combine(a_tbl, ta, tb)
    g1 = _sc_gather(c, s1)
    m1 = _edge_pass(g1, he1, w_i2, W_h, r2, False)
    g2 = _sc_gather(c, s2)
    m2 = _edge_pass(g2, he2, w_i2, W_h, r1, False)
    ma = _sc_scatter(m1, d1, zeros_tbl)
    mb = _sc_scatter(m2, d2, zeros_tbl)
    return _final(hv, ma, mb, w_o1, w_o2, b2, gid2)


# speedup vs baseline: 1.0595x; 1.0595x over previous
import functools

import jax
import jax.numpy as jnp
from jax import lax
from jax.experimental import pallas as pl
from jax.experimental.pallas import tpu as pltpu
from jax.experimental.pallas import tpu_sc as plsc

N_NODES = 10000
N_GRAPHS = 100
HID = 128
NC = 2
NS = 16
NW = NC * NS
E = 320000
EPT = E // NW
CHUNK = 80
NCHUNK = EPT // CHUNK
CBYTES = CHUNK * HID * 4
BE = 8000
NBLK = E // BE
HALF_BLKS = NBLK // 2

_PREC = lax.Precision.HIGHEST
_PREC_EDGE = lax.Precision.DEFAULT


@functools.cache
def _make_sc_gather():
    mesh = plsc.VectorSubcoreMesh(core_axis_name="c", subcore_axis_name="s",
                                  num_cores=NC, num_subcores=NS)

    @functools.partial(
        pl.kernel, mesh=mesh,
        out_type=jax.ShapeDtypeStruct((E, HID), jnp.float32),
        scratch_types=[
            pltpu.VMEM((NCHUNK, CHUNK), jnp.int32),
            pltpu.VMEM((CHUNK, HID), jnp.float32),
            pltpu.VMEM((CHUNK, HID), jnp.float32),
            pltpu.VMEM_SHARED((N_NODES, HID), jnp.float32),
            pltpu.SemaphoreType.DMA,
            pltpu.SemaphoreType.DMA,
            pltpu.SemaphoreType.DMA,
            pltpu.SemaphoreType.DMA,
        ],
    )
    def gather_k(table_hbm, idx_hbm, out_hbm,
                 idx_v, rows0, rows1, ctab, s0, s1, w0, w1):
        sid = lax.axis_index("s")
        wid = sid * NC + lax.axis_index("c")
        rows = (rows0, rows1)
        ssem = (s0, s1)
        wsem = (w0, w1)
        rpt = 624
        rem = N_NODES - NS * rpt
        pltpu.sync_copy(table_hbm.at[pl.ds(sid * rpt, rpt)],
                        ctab.at[pl.ds(sid * rpt, rpt)])

        @pl.when(sid == 0)
        def _():
            pltpu.sync_copy(table_hbm.at[pl.ds(NS * rpt, rem)],
                            ctab.at[pl.ds(NS * rpt, rem)])

        pltpu.sync_copy(idx_hbm.at[wid], idx_v)
        plsc.subcore_barrier()

        def out_slice(g):
            return out_hbm.at[pl.ds(wid * EPT + g * CHUNK, CHUNK)]

        def drain(sem, buf):
            pltpu.make_async_copy(out_hbm.at[pl.ds(0, CHUNK)], buf, sem).wait()

        pltpu.async_copy(ctab.at[idx_v.at[0]], rows[0], ssem[0])

        def body(t, _):
            for i, b in ((0, 1), (1, 0)):
                g = 2 * t + 1 + i
                if i == 0:
                    @pl.when(t > 0)
                    def _():
                        drain(wsem[b], rows[b])
                else:
                    drain(wsem[b], rows[b])
                pltpu.async_copy(ctab.at[idx_v.at[g]], rows[b], ssem[b])
                b1 = 1 - b
                drain(ssem[b1], rows[b1])
                pltpu.async_copy(rows[b1], out_slice(g - 1), wsem[b1])
            return 0

        lax.fori_loop(0, (NCHUNK - 1) // 2, body, 0)
        last = NCHUNK - 1
        drain(ssem[0], rows[0])
        pltpu.async_copy(rows[0], out_slice(last), wsem[0])
        drain(wsem[1], rows[1])
        drain(wsem[0], rows[0])

    return gather_k


@functools.cache
def _make_sc_scatter():
    mesh = plsc.VectorSubcoreMesh(core_axis_name="c", subcore_axis_name="s",
                                  num_cores=NC, num_subcores=NS)

    @functools.partial(
        pl.kernel, mesh=mesh,
        out_type=jax.ShapeDtypeStruct((NC, N_NODES, HID), jnp.float32),
        scratch_types=[
            pltpu.VMEM((NCHUNK, CHUNK), jnp.int32),
            pltpu.VMEM((CHUNK, HID), jnp.float32),
            pltpu.VMEM((CHUNK, HID), jnp.float32),
            pltpu.VMEM_SHARED((N_NODES, HID), jnp.float32),
            pltpu.SemaphoreType.DMA,
            pltpu.SemaphoreType.DMA,
            pltpu.SemaphoreType.DMA,
            pltpu.SemaphoreType.DMA,
        ],
    )
    def scatter_k(rows_hbm, idx_hbm, zero_hbm, out_hbm,
                  idx_v, rows0, rows1, shared, r0, r1, c0, c1):
        cid = lax.axis_index("c")
        sid = lax.axis_index("s")
        wid = sid * NC + cid
        rows = (rows0, rows1)
        rsem = (r0, r1)
        csem = (c0, c1)
        rpt = 624
        rem = N_NODES - NS * rpt
        pltpu.sync_copy(zero_hbm.at[pl.ds(sid * rpt, rpt)],
                        shared.at[pl.ds(sid * rpt, rpt)])

        @pl.when(sid == 0)
        def _():
            pltpu.sync_copy(zero_hbm.at[pl.ds(NS * rpt, rem)],
                            shared.at[pl.ds(NS * rpt, rem)])

        pltpu.sync_copy(idx_hbm.at[wid], idx_v)

        def in_slice(g):
            return rows_hbm.at[pl.ds(wid * EPT + g * CHUNK, CHUNK)]

        def drain(sem, buf):
            pltpu.make_async_copy(rows_hbm.at[pl.ds(0, CHUNK)], buf, sem).wait()

        pltpu.async_copy(in_slice(0), rows[0], rsem[0])
        plsc.subcore_barrier()

        def body(t, _):
            for i, b in ((0, 1), (1, 0)):
                g = 2 * t + 1 + i
                if i == 0:
                    @pl.when(t > 0)
                    def _():
                        drain(csem[b], rows[b])
                else:
                    drain(csem[b], rows[b])
                pltpu.async_copy(in_slice(g), rows[b], rsem[b])
                b1 = 1 - b
                drain(rsem[b1], rows[b1])
                pltpu.async_copy(rows[b1], shared.at[idx_v.at[g - 1]],
                                 csem[b1], add=True)
            return 0

        lax.fori_loop(0, (NCHUNK - 1) // 2, body, 0)
        last = NCHUNK - 1
        drain(rsem[0], rows[0])
        pltpu.async_copy(rows[0], shared.at[idx_v.at[last]], csem[0], add=True)
        drain(csem[1], rows[1])
        drain(csem[0], rows[0])
        plsc.subcore_barrier()
        pltpu.sync_copy(shared.at[pl.ds(sid * rpt, rpt)],
                        out_hbm.at[cid, pl.ds(sid * rpt, rpt)])

        @pl.when(sid == 0)
        def _():
            pltpu.sync_copy(shared.at[pl.ds(NS * rpt, rem)],
                            out_hbm.at[cid, pl.ds(NS * rpt, rem)])

    return scatter_k


def _sc_gather(table, idx2):
    return _make_sc_gather()(table, idx2)


def _sc_scatter(rows, idx2, zeros_tbl):
    return _make_sc_scatter()(rows, idx2, zeros_tbl)


def _node_matmul(x, w):
    def body(x_ref, w_ref, o_ref):
        o_ref[...] = jnp.dot(x_ref[...], w_ref[...],
                             preferred_element_type=jnp.float32,
                             precision=_PREC)
    return pl.pallas_call(
        body,
        out_shape=jax.ShapeDtypeStruct((x.shape[0], w.shape[1]), jnp.float32),
    )(x, w)


def _combine(a, tp):
    def body(a_ref, t_ref, o_ref):
        o_ref[...] = a_ref[...] + t_ref[0] + t_ref[1]
    return pl.pallas_call(
        body,
        out_shape=jax.ShapeDtypeStruct((N_NODES, HID), jnp.float32),
    )(a, tp)


def _edge_pass(g, he, w_i2, w_h, r_prev, emit_r):
    has_prev = r_prev is not None

    def body(*refs):
        if has_prev:
            g_ref, he_ref, wi_ref, wh_ref, r_ref, o_ref = refs
        else:
            g_ref, he_ref, wi_ref, wh_ref, o_ref = refs
        acc = g_ref[...] + jnp.dot(he_ref[...], wi_ref[...],
                                   preferred_element_type=jnp.float32,
                                   precision=_PREC_EDGE)
        if has_prev:
            acc = acc - r_ref[...]
        msg = jnp.maximum(acc, 0.0)
        if emit_r:
            o_ref[...] = jnp.dot(msg, wh_ref[...],
                                 preferred_element_type=jnp.float32,
                                 precision=_PREC_EDGE)
        else:
            o_ref[...] = msg

    in_specs = [
        pl.BlockSpec((BE, HID), lambda b: (b, 0)),
        pl.BlockSpec((BE, 16), lambda b: (b, 0)),
        pl.BlockSpec((16, HID), lambda b: (0, 0)),
        pl.BlockSpec((HID, HID), lambda b: (0, 0)),
    ]
    args = [g, he, w_i2, w_h]
    if has_prev:
        in_specs.append(pl.BlockSpec((BE, HID),
                                     lambda b: ((b + HALF_BLKS) % NBLK, 0)))
        args.append(r_prev)
    return pl.pallas_call(
        body,
        grid=(NBLK,),
        in_specs=in_specs,
        out_specs=pl.BlockSpec((BE, HID), lambda b: (b, 0)),
        out_shape=jax.ShapeDtypeStruct((E, HID), jnp.float32),
    )(*args)


def _final(hv, mp, w_o1, w_o2, b_o, gids):
    def body(hv_ref, mp_ref, w1_ref, w2_ref, b_ref, gid_ref, o_ref):
        m = mp_ref[0] + mp_ref[1]
        h = jnp.dot(hv_ref[...], w1_ref[...],
                    preferred_element_type=jnp.float32, precision=_PREC)
        h += jnp.dot(m, w2_ref[...],
                     preferred_element_type=jnp.float32, precision=_PREC)
        h = jnp.maximum(h + b_ref[...], 0.0)
        rows = lax.broadcasted_iota(jnp.int32, (N_GRAPHS, N_NODES), 0)
        p = (gid_ref[...] == rows).astype(jnp.float32)
        g_sum = jnp.dot(p, h, preferred_element_type=jnp.float32,
                        precision=_PREC)
        counts = jnp.maximum(jnp.sum(p, axis=1, keepdims=True), 1.0)
        o_ref[...] = g_sum / counts

    return pl.pallas_call(
        body,
        out_shape=jax.ShapeDtypeStruct((N_GRAPHS, HID), jnp.float32),
    )(hv, mp, w_o1, w_o2, b_o, gids)


def kernel(hv, edge_index, he, graph_ids, W_i, W_h, W_o, b_o):
    u, v = edge_index[0], edge_index[1]
    src = jnp.concatenate([u, v])
    dst = jnp.concatenate([v, u])
    src2 = src.reshape(NW, NCHUNK, CHUNK)
    dst2 = dst.reshape(NW, NCHUNK, CHUNK)
    zeros_tbl = jnp.zeros((N_NODES, HID), jnp.float32)

    atom = hv.shape[1]
    w_i1 = W_i[:atom]
    w_i2 = W_i[atom:]
    w_o1 = W_o[:atom]
    w_o2 = W_o[atom:]
    b2 = b_o.reshape(1, HID)
    gid2 = graph_ids.reshape(1, N_NODES)

    a_tbl = _node_matmul(hv, w_i1)

    g = _sc_gather(a_tbl, src2)
    r = _edge_pass(g, he, w_i2, W_h, None, True)
    for _ in range(4):
        tp = _sc_scatter(r, dst2, zeros_tbl)
        c = _combine(a_tbl, tp)
        g = _sc_gather(c, src2)
        r = _edge_pass(g, he, w_i2, W_h, r, True)
    tp = _sc_scatter(r, dst2, zeros_tbl)
    c = _combine(a_tbl, tp)
    g = _sc_gather(c, src2)
    msg = _edge_pass(g, he, w_i2, W_h, r, False)
    mp = _sc_scatter(msg, dst2, zeros_tbl)
    return _final(hv, mp, w_o1, w_o2, b2, gid2)
